# baseline (speedup 1.0000x reference)
import functools

import jax
import jax.numpy as jnp
from jax import lax
from jax.experimental import pallas as pl
from jax.experimental.pallas import tpu as pltpu
from jax.experimental.pallas import tpu_sc as plsc

B = 16384
ED = 64
EPS = 1e-5

NC = 2
NS = 16
NW = NC * NS
BPW = B // NW
CHUNK = 128
NCH = BPW // CHUNK
NG = BPW // 16


def _stage(idx_v, row_v, half_v):
    for j in range(NCH):
        for g in range(CHUNK // 16):
            v = idx_v[j, pl.ds(g * 16, 16)]
            row_v[pl.ds(j * CHUNK + g * 16, 16)] = v >> 1
            half_v[pl.ds(j * CHUNK + g * 16, 16)] = v & 1


def _gather_halves(tbl_hbm, row_v, half_v, rows_v, out_v, sem):
    for h in range(2):
        for j in range(2):
            off = h * 256 + j * CHUNK
            pltpu.async_copy(tbl_hbm.at[row_v.at[pl.ds(off, CHUNK)]],
                             rows_v.at[pl.ds(j * CHUNK, CHUNK)], sem)
        for j in range(2):
            off = h * 256 + j * CHUNK
            pltpu.make_async_copy(tbl_hbm.at[row_v.at[pl.ds(off, CHUNK)]],
                                  rows_v.at[pl.ds(j * CHUNK, CHUNK)],
                                  sem).wait()

        def body(g, carry):
            rid = g * 16 + lax.iota(jnp.int32, 16)
            rid_out = h * 256 + rid
            cbase = half_v[pl.ds(h * 256 + g * 16, 16)] * 64
            for c in range(ED):
                val = plsc.load_gather(rows_v, [rid, cbase + c])
                plsc.store_scatter(out_v,
                                   [rid_out, jnp.full((16,), c, jnp.int32)],
                                   val)
            return carry

        lax.fori_loop(0, 256 // 16, body, 0)


def _gather_body(uid_hbm, mid_hbm, ut_hbm, mt_hbm, ue_out, me_out,
                 uidx_v, midx_v, urow_v, mrow_v, uhalf_v, mhalf_v,
                 rows_v, out_v, sem):
    wid = lax.axis_index("s") * NC + lax.axis_index("c")
    base = wid * BPW
    pltpu.sync_copy(uid_hbm.at[wid], uidx_v)
    pltpu.sync_copy(mid_hbm.at[wid], midx_v)
    _stage(uidx_v, urow_v, uhalf_v)
    _stage(midx_v, mrow_v, mhalf_v)
    _gather_halves(ut_hbm, urow_v, uhalf_v, rows_v, out_v, sem)
    pltpu.sync_copy(out_v, ue_out.at[pl.ds(base, BPW)])
    _gather_halves(mt_hbm, mrow_v, mhalf_v, rows_v, out_v, sem)
    pltpu.sync_copy(out_v, me_out.at[pl.ds(base, BPW)])


@functools.cache
def _make_gather():
    return pl.kernel(
        _gather_body,
        out_type=[
            jax.ShapeDtypeStruct((B, ED), jnp.float32),
            jax.ShapeDtypeStruct((B, ED), jnp.float32),
        ],
        mesh=plsc.VectorSubcoreMesh(
            core_axis_name="c", subcore_axis_name="s",
            num_cores=NC, num_subcores=NS),
        compiler_params=pltpu.CompilerParams(needs_layout_passes=False),
        scratch_types=[
            pltpu.VMEM((NCH, CHUNK), jnp.int32),
            pltpu.VMEM((NCH, CHUNK), jnp.int32),
            pltpu.VMEM((BPW,), jnp.int32),
            pltpu.VMEM((BPW,), jnp.int32),
            pltpu.VMEM((BPW,), jnp.int32),
            pltpu.VMEM((BPW,), jnp.int32),
            pltpu.VMEM((BPW // 2, 2 * ED), jnp.float32),
            pltpu.VMEM((BPW, ED), jnp.float32),
            pltpu.SemaphoreType.DMA,
        ],
    )


def _bn(x, g, be):
    mu = jnp.mean(x, axis=0, keepdims=True)
    d = x - mu
    var = jnp.mean(d * d, axis=0, keepdims=True)
    return d * lax.rsqrt(var + EPS) * g + be


def _mlp_body(ue, me, w1a, w1b, b1, g1, be1, w2, b2, g2, be2,
              w3, b3, g3, be3, w4, b4, out):
    f32 = jnp.float32
    x = (jnp.dot(ue[...], w1a[...], preferred_element_type=f32)
         + jnp.dot(me[...], w1b[...], preferred_element_type=f32)
         + b1[...])
    x = _bn(jnp.maximum(x, 0.0), g1[...], be1[...])
    x = jnp.dot(x, w2[...], preferred_element_type=f32) + b2[...]
    x = _bn(jnp.maximum(x, 0.0), g2[...], be2[...])
    x = jnp.dot(x, w3[...], preferred_element_type=f32) + b3[...]
    x = _bn(jnp.maximum(x, 0.0), g3[...], be3[...])
    logit = jnp.dot(x, w4[...], preferred_element_type=f32) + b4[...]
    out[...] = jax.nn.sigmoid(logit) * 4.5 + 0.5


_mlp = pl.pallas_call(
    _mlp_body,
    out_shape=jax.ShapeDtypeStruct((B, 1), jnp.float32),
)


def kernel(user_ids, movie_ids, user_table, movie_table,
           W1, b1, g1, be1, W2, b2, g2, be2, W3, b3, g3, be3, W4, b4):
    uid = user_ids.astype(jnp.int32).reshape(NW, NCH, CHUNK)
    mid = movie_ids.astype(jnp.int32).reshape(NW, NCH, CHUNK)
    ue, me = _make_gather()(uid, mid,
                            user_table.reshape(-1, 2 * ED),
                            movie_table.reshape(-1, 2 * ED))
    row = lambda v: v.reshape(1, -1)
    out = _mlp(ue, me, W1[:ED], W1[ED:],
               row(b1), row(g1), row(be1),
               W2, row(b2), row(g2), row(be2),
               W3, row(b3), row(g3), row(be3),
               W4, b4.reshape(1, 1))
    return out.reshape(B)


# speedup vs baseline: 1.0966x; 1.0966x over previous
import functools

import jax
import jax.numpy as jnp
from jax import lax
from jax.experimental import pallas as pl
from jax.experimental.pallas import tpu as pltpu
from jax.experimental.pallas import tpu_sc as plsc

B = 16384
ED = 64
EPS = 1e-5

NC = 2
NS = 16
NW = NC * NS
BPW = B // NW
CHUNK = 128
NCH = BPW // CHUNK


def _gather_body(uid_hbm, mid_hbm, ut_hbm, mt_hbm, ue_out, me_out,
                 uidx_v, midx_v, urows_v, mrows_v, sem):
    wid = lax.axis_index("s") * NC + lax.axis_index("c")
    base = wid * BPW
    pltpu.sync_copy(uid_hbm.at[wid], uidx_v)
    pltpu.sync_copy(mid_hbm.at[wid], midx_v)
    for j in range(NCH):
        pltpu.async_copy(ut_hbm.at[uidx_v.at[j]],
                         urows_v.at[pl.ds(j * CHUNK, CHUNK)], sem)
    for j in range(NCH):
        pltpu.async_copy(mt_hbm.at[midx_v.at[j]],
                         mrows_v.at[pl.ds(j * CHUNK, CHUNK)], sem)
    for j in range(NCH):
        pltpu.make_async_copy(ut_hbm.at[uidx_v.at[j]],
                              urows_v.at[pl.ds(j * CHUNK, CHUNK)], sem).wait()
    for j in range(NCH):
        pltpu.make_async_copy(mt_hbm.at[midx_v.at[j]],
                              mrows_v.at[pl.ds(j * CHUNK, CHUNK)], sem).wait()
    pltpu.sync_copy(urows_v, ue_out.at[pl.ds(base, BPW)])
    pltpu.sync_copy(mrows_v, me_out.at[pl.ds(base, BPW)])


@functools.cache
def _make_gather():
    return pl.kernel(
        _gather_body,
        out_type=[
            jax.ShapeDtypeStruct((B, ED), jnp.float32),
            jax.ShapeDtypeStruct((B, ED), jnp.float32),
        ],
        mesh=plsc.VectorSubcoreMesh(
            core_axis_name="c", subcore_axis_name="s",
            num_cores=NC, num_subcores=NS),
        compiler_params=pltpu.CompilerParams(
            needs_layout_passes=False, use_tc_tiling_on_sc=False),
        scratch_types=[
            pltpu.VMEM((NCH, CHUNK), jnp.int32),
            pltpu.VMEM((NCH, CHUNK), jnp.int32),
            pltpu.VMEM((BPW, ED), jnp.float32),
            pltpu.VMEM((BPW, ED), jnp.float32),
            pltpu.SemaphoreType.DMA,
        ],
    )


def _bn(x, g, be):
    mu = jnp.mean(x, axis=0, keepdims=True)
    d = x - mu
    var = jnp.mean(d * d, axis=0, keepdims=True)
    return d * lax.rsqrt(var + EPS) * g + be


def _mlp_body(ue, me, w1a, w1b, b1, g1, be1, w2, b2, g2, be2,
              w3, b3, g3, be3, w4, b4, out):
    f32 = jnp.float32
    x = (jnp.dot(ue[...], w1a[...], preferred_element_type=f32)
         + jnp.dot(me[...], w1b[...], preferred_element_type=f32)
         + b1[...])
    x = _bn(jnp.maximum(x, 0.0), g1[...], be1[...])
    x = jnp.dot(x, w2[...], preferred_element_type=f32) + b2[...]
    x = _bn(jnp.maximum(x, 0.0), g2[...], be2[...])
    x = jnp.dot(x, w3[...], preferred_element_type=f32) + b3[...]
    x = _bn(jnp.maximum(x, 0.0), g3[...], be3[...])
    logit = jnp.dot(x, w4[...], preferred_element_type=f32) + b4[...]
    out[...] = jax.nn.sigmoid(logit) * 4.5 + 0.5


_mlp = pl.pallas_call(
    _mlp_body,
    out_shape=jax.ShapeDtypeStruct((B, 1), jnp.float32),
)


def kernel(user_ids, movie_ids, user_table, movie_table,
           W1, b1, g1, be1, W2, b2, g2, be2, W3, b3, g3, be3, W4, b4):
    uid = user_ids.astype(jnp.int32).reshape(NW, NCH, CHUNK)
    mid = movie_ids.astype(jnp.int32).reshape(NW, NCH, CHUNK)
    ue, me = _make_gather()(uid, mid, user_table, movie_table)
    row = lambda v: v.reshape(1, -1)
    out = _mlp(ue, me, W1[:ED], W1[ED:],
               row(b1), row(g1), row(be1),
               W2, row(b2), row(g2), row(be2),
               W3, row(b3), row(g3), row(be3),
               W4, b4.reshape(1, 1))
    return out.reshape(B)
